# baseline (speedup 1.0000x reference)
import functools

import jax
import jax.numpy as jnp
from jax import lax
from jax.experimental import pallas as pl
from jax.experimental.pallas import tpu as pltpu
from jax.experimental.pallas import tpu_sc as plsc

_N = 20480
_PRE = 6000
_CAP = 6144 + 20480
_TRASH = 6144
_THR = 0.3
_K = 300
_NEG = float("-inf")
_LANES = 128
_SUB = 8
_NW = 32
_CHK = 128
_PER = _N // _NW
_VEC = 16


def _thresh_body(scores_ref, dest_ref):
    R = scores_ref.shape[0]
    scores = scores_ref[:]
    sbits = lax.bitcast_convert_type(scores, jnp.int32)
    row = lax.broadcasted_iota(jnp.int32, (R, _LANES), 0)
    col = lax.broadcasted_iota(jnp.int32, (R, _LANES), 1)
    flat = row * _LANES + col

    def bs_val(_, lh):
        lo, hi = lh
        mid = lo + (hi - lo) // 2
        below = jnp.sum((sbits > mid).astype(jnp.int32)) < _PRE
        return (jnp.where(below, lo, mid + 1), jnp.where(below, mid, hi))

    tbits, _ = lax.fori_loop(0, 31, bs_val,
                             (jnp.int32(0), jnp.int32(0x3F800000)))
    gt = sbits > tbits
    eq = sbits == tbits
    need = _PRE - jnp.sum(gt.astype(jnp.int32))

    def bs_idx(_, lh):
        lo, hi = lh
        mid = lo + (hi - lo) // 2
        ok = jnp.sum((eq & (flat < mid)).astype(jnp.int32)) >= need
        return (jnp.where(ok, lo, mid + 1), jnp.where(ok, mid, hi))

    cut, _ = lax.fori_loop(0, 16, bs_idx,
                           (jnp.int32(0), jnp.int32(R * _LANES)))
    elig = gt | (eq & (flat < cut))

    eli = jnp.where(elig, 1, 0).astype(jnp.int32)
    x = eli
    for k in (1, 2, 4, 8, 16, 32, 64):
        x = x + jnp.where(col >= k, pltpu.roll(x, k, 1), 0)
    rowtot = jnp.max(x, axis=1, keepdims=True)
    s = rowtot
    for k in (1, 2, 4, 8, 16, 32, 64, 128):
        s = s + jnp.where(row[:, 0:1] >= k, pltpu.roll(s, k, 0), 0)
    dest = (s - rowtot) + (x - eli)
    dest_ref[:] = jnp.where(elig, dest, _TRASH + flat)


def _sc_compact_body(p0, p1, p2, p3, p4, p5, p6, dest_hbm,
                     o0, o1, o2, o3, o4, o5, o6,
                     v0, v1, v2, v3, v4, v5, v6, i0, i1, i2, i3, i4, sem):
    cid = lax.axis_index("c")
    sid = lax.axis_index("s")
    wid = cid * 16 + sid
    base = wid * _PER
    planes = (p0, p1, p2, p3, p4, p5, p6)
    outs = (o0, o1, o2, o3, o4, o5, o6)
    vals = (v0, v1, v2, v3, v4, v5, v6)
    idxs = (i0, i1, i2, i3, i4)
    nch = _PER // _CHK

    loads = [pltpu.async_copy(dest_hbm.at[pl.ds(base + j * _CHK, _CHK)],
                              idxs[j], sem) for j in range(nch)]
    loads += [pltpu.async_copy(planes[k].at[pl.ds(base, _PER)], vals[k], sem)
              for k in range(7)]
    for h in loads:
        h.wait()
    stores = [pltpu.async_copy(vals[k].at[pl.ds(j * _CHK, _CHK)],
                               outs[k].at[idxs[j]], sem)
              for k in range(7) for j in range(nch)]
    for h in stores:
        h.wait()


def _sc_compact(planes, dest):
    mesh = plsc.VectorSubcoreMesh(core_axis_name="c", subcore_axis_name="s")
    f = pl.kernel(
        _sc_compact_body,
        mesh=mesh,
        out_type=[jax.ShapeDtypeStruct((_CAP,), jnp.float32)] * 7,
        scratch_types=[pltpu.VMEM((_PER,), jnp.float32)] * 7
        + [pltpu.VMEM((_CHK,), jnp.int32)] * (_PER // _CHK)
        + [pltpu.SemaphoreType.DMA],
    )
    return f(*[planes[k] for k in range(7)], dest)


def _nms_body(scores_ref, coords_ref, out_ref, ws_ref, vols_ref):
    R = scores_ref.shape[0]
    NCH = R // _SUB
    big = jnp.int32(2 ** 30)
    ws_ref[:] = scores_ref[:]
    vols_ref[:] = ((coords_ref[3] - coords_ref[0])
                   * (coords_ref[4] - coords_ref[1])
                   * (coords_ref[5] - coords_ref[2]))

    ii = (lax.broadcasted_iota(jnp.int32, (_SUB, _LANES), 0) * _LANES
          + lax.broadcasted_iota(jnp.int32, (_SUB, _LANES), 1))
    lane = lax.broadcasted_iota(jnp.int32, (1, _LANES), 1)

    def comb_ordered(acc, cand):
        if acc is None:
            return cand
        c = acc[0] >= cand[0]
        return tuple(jnp.where(c, a, b) for a, b in zip(acc, cand))

    def comb_lex(a, b):
        c = (a[0] > b[0]) | ((a[0] == b[0]) & (a[1] < b[1]))
        return tuple(jnp.where(c, x, y) for x, y in zip(a, b))

    def fold_rows(t):
        h = 4
        while h >= 1:
            t = comb_lex(tuple(x[:h] for x in t), tuple(x[h:] for x in t))
            h //= 2
        return t

    def bf_max(x):
        for k in (1, 2, 4, 8, 16, 32, 64):
            x = jnp.maximum(x, pltpu.roll(x, k, 1))
        return x

    def bf_min(x):
        for k in (1, 2, 4, 8, 16, 32, 64):
            x = jnp.minimum(x, pltpu.roll(x, k, 1))
        return x

    def load_chunk(c):
        sl = pl.ds(c * _SUB, _SUB)
        return tuple(coords_ref[k, sl, :] for k in range(6))

    def winner(carry):
        v1, i1 = carry[0], carry[1]
        mv = bf_max(v1)
        selv = bf_min(jnp.where(v1 == mv, i1, big))
        wm = (v1 == mv) & (i1 == selv)
        coords = tuple(bf_max(jnp.where(wm, carry[2 + k], _NEG))
                       for k in range(6))
        return mv, coords

    accs = [None, None]
    for c in range(NCH):
        w = ws_ref[pl.ds(c * _SUB, _SUB), :]
        cand = (w, ii + c * _SUB * _LANES) + load_chunk(c)
        accs[c & 1] = comb_ordered(accs[c & 1], cand)
    carry0 = fold_rows(comb_lex(accs[0], accs[1]))
    m0, coords0 = winner(carry0)

    def body(i, carry):
        m, coords = winner(carry)
        empty = m == _NEG
        sc = jnp.where(empty, m0, m)
        y1, x1, z1, y2, x2, z2 = (
            jnp.where(empty, c0, c) for c0, c in zip(coords0, coords))
        vol1 = (y2 - y1) * (x2 - x1) * (z2 - z1)

        accs = [None, None]
        for c in range(NCH):
            sl = pl.ds(c * _SUB, _SUB)
            w = ws_ref[sl, :]
            b0, b1, b2, b3, b4, b5 = load_chunk(c)
            vv = vols_ref[sl, :]
            inter = (jnp.maximum(jnp.minimum(y2, b3) - jnp.maximum(y1, b0), 0.0)
                     * jnp.maximum(jnp.minimum(x2, b4) - jnp.maximum(x1, b1), 0.0)
                     * jnp.maximum(jnp.minimum(z2, b5) - jnp.maximum(z1, b2), 0.0))
            iou = inter / ((vol1 + vv - inter) + 1e-8)
            neww = jnp.where(iou > _THR, _NEG, w)
            ws_ref[sl, :] = neww
            cand = (neww, ii + c * _SUB * _LANES, b0, b1, b2, b3, b4, b5)
            accs[c & 1] = comb_ordered(accs[c & 1], cand)

        v = jnp.full((1, _LANES), 0.0, jnp.float32)
        for j, val in enumerate((y1, x1, z1, y2, x2, z2, sc)):
            v = jnp.where(lane == j, val, v)
        out_ref[pl.ds(i, 1), :] = v
        return fold_rows(comb_lex(accs[0], accs[1]))

    lax.fori_loop(0, _K, body, carry0)


@jax.jit
def kernel(boxes, scores):
    n = scores.shape[0]
    pad = _N - n
    scores_p = jnp.concatenate(
        [scores, jnp.full((pad,), _NEG, jnp.float32)])
    boxes_p = jnp.concatenate([boxes, jnp.zeros((pad, 6), jnp.float32)],
                              axis=0)
    planes = list(boxes_p.T) + [scores_p]

    dest_plane = pl.pallas_call(
        _thresh_body,
        out_shape=jax.ShapeDtypeStruct((_N // _LANES, _LANES), jnp.int32),
    )(scores_p.reshape(_N // _LANES, _LANES))

    comp = _sc_compact(planes, dest_plane.reshape(_N))
    keep = lax.broadcasted_iota(jnp.int32, (_TRASH,), 0) < _PRE
    scores_c = jnp.where(keep, comp[6][:_TRASH], _NEG).reshape(
        _TRASH // _LANES, _LANES)
    coords_c = jnp.stack(
        [jnp.where(keep, comp[k][:_TRASH], 0.0).reshape(
            _TRASH // _LANES, _LANES) for k in range(6)])

    out = pl.pallas_call(
        _nms_body,
        out_shape=jax.ShapeDtypeStruct((304, _LANES), jnp.float32),
        scratch_shapes=[
            pltpu.VMEM((_TRASH // _LANES, _LANES), jnp.float32),
            pltpu.VMEM((_TRASH // _LANES, _LANES), jnp.float32),
        ],
    )(scores_c, coords_c)
    return out[:_K, :7]


# speedup vs baseline: 1.6740x; 1.6740x over previous
import functools

import jax
import jax.numpy as jnp
from jax import lax
from jax.experimental import pallas as pl
from jax.experimental.pallas import tpu as pltpu
from jax.experimental.pallas import tpu_sc as plsc

_N = 20480
_PRE = 6000
_CAP = 6144 + 20480
_TRASH = 6144
_THR = 0.3
_K = 300
_NEG = float("-inf")
_LANES = 128
_SUB = 8
_NW = 32
_CHK = 128
_PER = _N // 16
_VEC = 16


def _thresh_body(scores_ref, dest_ref):
    R = scores_ref.shape[0]
    scores = scores_ref[:]
    sbits = lax.bitcast_convert_type(scores, jnp.int32)
    row = lax.broadcasted_iota(jnp.int32, (R, _LANES), 0)
    col = lax.broadcasted_iota(jnp.int32, (R, _LANES), 1)
    flat = row * _LANES + col

    def bs_val(_, lh):
        lo, hi = lh
        mid = lo + (hi - lo) // 2
        below = jnp.sum((sbits > mid).astype(jnp.int32)) < _PRE
        return (jnp.where(below, lo, mid + 1), jnp.where(below, mid, hi))

    tbits, _ = lax.fori_loop(0, 31, bs_val,
                             (jnp.int32(0), jnp.int32(0x3F800000)))
    gt = sbits > tbits
    eq = sbits == tbits
    need = _PRE - jnp.sum(gt.astype(jnp.int32))

    def bs_idx(_, lh):
        lo, hi = lh
        mid = lo + (hi - lo) // 2
        ok = jnp.sum((eq & (flat < mid)).astype(jnp.int32)) >= need
        return (jnp.where(ok, lo, mid + 1), jnp.where(ok, mid, hi))

    cut, _ = lax.fori_loop(0, 16, bs_idx,
                           (jnp.int32(0), jnp.int32(R * _LANES)))
    elig = gt | (eq & (flat < cut))

    eli = jnp.where(elig, 1, 0).astype(jnp.int32)
    x = eli
    for k in (1, 2, 4, 8, 16, 32, 64):
        x = x + jnp.where(col >= k, pltpu.roll(x, k, 1), 0)
    rowtot = jnp.max(x, axis=1, keepdims=True)
    s = rowtot
    for k in (1, 2, 4, 8, 16, 32, 64, 128):
        s = s + jnp.where(row[:, 0:1] >= k, pltpu.roll(s, k, 0), 0)
    dest = (s - rowtot) + (x - eli)
    dest_ref[:] = jnp.where(elig, dest, _TRASH + flat)


def _sc_compact_body(p0, p1, p2, p3, p4, p5, p6, dest_hbm,
                     o0, o1, o2, o3, o4, o5, o6,
                     v0, v1, v2, v3, v4, v5, v6,
                     i0, i1, i2, i3, i4, i5, i6, i7, i8, i9,
                     s0, s1, s2, s3, s4, s5, s6, sem):
    cid = lax.axis_index("c")
    sid = lax.axis_index("s")
    active = cid == 0
    base = sid * _PER
    planes = (p0, p1, p2, p3, p4, p5, p6)
    outs = (o0, o1, o2, o3, o4, o5, o6)
    vals = (v0, v1, v2, v3, v4, v5, v6)
    shared = (s0, s1, s2, s3, s4, s5, s6)
    idxs = (i0, i1, i2, i3, i4, i5, i6, i7, i8, i9)
    nch = _PER // _CHK
    slc = _TRASH // 16

    @pl.when(active)
    def _():
        loads = [pltpu.async_copy(dest_hbm.at[pl.ds(base + j * _CHK, _CHK)],
                                  idxs[j], sem) for j in range(nch)]
        loads += [pltpu.async_copy(planes[k].at[pl.ds(base, _PER)], vals[k],
                                   sem) for k in range(7)]
        for h in loads:
            h.wait()
        stores = [pltpu.async_copy(vals[k].at[pl.ds(j * _CHK, _CHK)],
                                   shared[k].at[idxs[j]], sem)
                  for k in range(7) for j in range(nch)]
        for h in stores:
            h.wait()

    plsc.subcore_barrier()

    @pl.when(active)
    def _():
        outc = [pltpu.async_copy(shared[k].at[pl.ds(sid * slc, slc)],
                                 outs[k].at[pl.ds(sid * slc, slc)], sem)
                for k in range(7)]
        for h in outc:
            h.wait()


def _sc_compact(planes, dest):
    mesh = plsc.VectorSubcoreMesh(core_axis_name="c", subcore_axis_name="s")
    f = pl.kernel(
        _sc_compact_body,
        mesh=mesh,
        out_type=[jax.ShapeDtypeStruct((_TRASH,), jnp.float32)] * 7,
        scratch_types=[pltpu.VMEM((_PER,), jnp.float32)] * 7
        + [pltpu.VMEM((_CHK,), jnp.int32)] * (_PER // _CHK)
        + [pltpu.VMEM_SHARED((_CAP,), jnp.float32)] * 7
        + [pltpu.SemaphoreType.DMA],
    )
    return f(*[planes[k] for k in range(7)], dest)


def _nms_body(scores_ref, coords_ref, out_ref, ws_ref, vols_ref):
    R = scores_ref.shape[0]
    NCH = R // _SUB
    big = jnp.int32(2 ** 30)
    ws_ref[:] = scores_ref[:]
    vols_ref[:] = ((coords_ref[3] - coords_ref[0])
                   * (coords_ref[4] - coords_ref[1])
                   * (coords_ref[5] - coords_ref[2]))

    ii = (lax.broadcasted_iota(jnp.int32, (_SUB, _LANES), 0) * _LANES
          + lax.broadcasted_iota(jnp.int32, (_SUB, _LANES), 1))
    lane = lax.broadcasted_iota(jnp.int32, (1, _LANES), 1)

    def comb_ordered(acc, cand):
        if acc is None:
            return cand
        c = acc[0] >= cand[0]
        return tuple(jnp.where(c, a, b) for a, b in zip(acc, cand))

    def comb_lex(a, b):
        c = (a[0] > b[0]) | ((a[0] == b[0]) & (a[1] < b[1]))
        return tuple(jnp.where(c, x, y) for x, y in zip(a, b))

    def fold_rows(t):
        h = 4
        while h >= 1:
            t = comb_lex(tuple(x[:h] for x in t), tuple(x[h:] for x in t))
            h //= 2
        return t

    def bf_max(x):
        for k in (1, 2, 4, 8, 16, 32, 64):
            x = jnp.maximum(x, pltpu.roll(x, k, 1))
        return x

    def bf_min(x):
        for k in (1, 2, 4, 8, 16, 32, 64):
            x = jnp.minimum(x, pltpu.roll(x, k, 1))
        return x

    def load_chunk(c):
        sl = pl.ds(c * _SUB, _SUB)
        return tuple(coords_ref[k, sl, :] for k in range(6))

    def winner(carry):
        v1, i1 = carry[0], carry[1]
        mv = bf_max(v1)
        selv = bf_min(jnp.where(v1 == mv, i1, big))
        wm = (v1 == mv) & (i1 == selv)
        coords = tuple(bf_max(jnp.where(wm, carry[2 + k], _NEG))
                       for k in range(6))
        return mv, coords

    accs = [None, None]
    for c in range(NCH):
        w = ws_ref[pl.ds(c * _SUB, _SUB), :]
        cand = (w, ii + c * _SUB * _LANES) + load_chunk(c)
        accs[c & 1] = comb_ordered(accs[c & 1], cand)
    carry0 = fold_rows(comb_lex(accs[0], accs[1]))
    m0, coords0 = winner(carry0)

    def body(i, carry):
        m, coords = winner(carry)
        empty = m == _NEG
        sc = jnp.where(empty, m0, m)
        y1, x1, z1, y2, x2, z2 = (
            jnp.where(empty, c0, c) for c0, c in zip(coords0, coords))
        vol1 = (y2 - y1) * (x2 - x1) * (z2 - z1)

        accs = [None, None]
        for c in range(NCH):
            sl = pl.ds(c * _SUB, _SUB)
            w = ws_ref[sl, :]
            b0, b1, b2, b3, b4, b5 = load_chunk(c)
            vv = vols_ref[sl, :]
            inter = (jnp.maximum(jnp.minimum(y2, b3) - jnp.maximum(y1, b0), 0.0)
                     * jnp.maximum(jnp.minimum(x2, b4) - jnp.maximum(x1, b1), 0.0)
                     * jnp.maximum(jnp.minimum(z2, b5) - jnp.maximum(z1, b2), 0.0))
            iou = inter / ((vol1 + vv - inter) + 1e-8)
            neww = jnp.where(iou > _THR, _NEG, w)
            ws_ref[sl, :] = neww
            cand = (neww, ii + c * _SUB * _LANES, b0, b1, b2, b3, b4, b5)
            accs[c & 1] = comb_ordered(accs[c & 1], cand)

        v = jnp.full((1, _LANES), 0.0, jnp.float32)
        for j, val in enumerate((y1, x1, z1, y2, x2, z2, sc)):
            v = jnp.where(lane == j, val, v)
        out_ref[pl.ds(i, 1), :] = v
        return fold_rows(comb_lex(accs[0], accs[1]))

    lax.fori_loop(0, _K, body, carry0)


@jax.jit
def kernel(boxes, scores):
    n = scores.shape[0]
    pad = _N - n
    scores_p = jnp.concatenate(
        [scores, jnp.full((pad,), _NEG, jnp.float32)])
    boxes_p = jnp.concatenate([boxes, jnp.zeros((pad, 6), jnp.float32)],
                              axis=0)
    planes = list(boxes_p.T) + [scores_p]

    dest_plane = pl.pallas_call(
        _thresh_body,
        out_shape=jax.ShapeDtypeStruct((_N // _LANES, _LANES), jnp.int32),
    )(scores_p.reshape(_N // _LANES, _LANES))

    comp = _sc_compact(planes, dest_plane.reshape(_N))
    keep = lax.broadcasted_iota(jnp.int32, (_TRASH,), 0) < _PRE
    scores_c = jnp.where(keep, comp[6], _NEG).reshape(
        _TRASH // _LANES, _LANES)
    coords_c = jnp.stack(
        [jnp.where(keep, comp[k], 0.0).reshape(
            _TRASH // _LANES, _LANES) for k in range(6)])

    out = pl.pallas_call(
        _nms_body,
        out_shape=jax.ShapeDtypeStruct((304, _LANES), jnp.float32),
        scratch_shapes=[
            pltpu.VMEM((_TRASH // _LANES, _LANES), jnp.float32),
            pltpu.VMEM((_TRASH // _LANES, _LANES), jnp.float32),
        ],
    )(scores_c, coords_c)
    return out[:_K, :7]


# speedup vs baseline: 4.8259x; 2.8828x over previous
import functools

import jax
import jax.numpy as jnp
from jax import lax
from jax.experimental import pallas as pl
from jax.experimental.pallas import tpu as pltpu
from jax.experimental.pallas import tpu_sc as plsc

_N = 20480
_PRE = 6000
_CAP = 6144 + 20480
_TRASH = 6144
_THR = 0.3
_K = 300
_NEG = float("-inf")
_LANES = 128
_SUB = 8
_NW = 32
_CHK = 128
_PER = _N // 16
_VEC = 16


def _thresh_body(scores_ref, dest_ref):
    R = scores_ref.shape[0]
    scores = scores_ref[:]
    sbits = lax.bitcast_convert_type(scores, jnp.int32)
    row = lax.broadcasted_iota(jnp.int32, (R, _LANES), 0)
    col = lax.broadcasted_iota(jnp.int32, (R, _LANES), 1)
    flat = row * _LANES + col

    def bs_val(_, lh):
        lo, hi = lh
        mid = lo + (hi - lo) // 2
        below = jnp.sum((sbits > mid).astype(jnp.int32)) < _PRE
        return (jnp.where(below, lo, mid + 1), jnp.where(below, mid, hi))

    tbits, _ = lax.fori_loop(0, 31, bs_val,
                             (jnp.int32(0), jnp.int32(0x3F800000)))
    gt = sbits > tbits
    eq = sbits == tbits
    need = _PRE - jnp.sum(gt.astype(jnp.int32))

    def bs_idx(_, lh):
        lo, hi = lh
        mid = lo + (hi - lo) // 2
        ok = jnp.sum((eq & (flat < mid)).astype(jnp.int32)) >= need
        return (jnp.where(ok, lo, mid + 1), jnp.where(ok, mid, hi))

    cut, _ = lax.fori_loop(0, 16, bs_idx,
                           (jnp.int32(0), jnp.int32(R * _LANES)))
    elig = gt | (eq & (flat < cut))

    eli = jnp.where(elig, 1, 0).astype(jnp.int32)
    x = eli
    for k in (1, 2, 4, 8, 16, 32, 64):
        x = x + jnp.where(col >= k, pltpu.roll(x, k, 1), 0)
    rowtot = jnp.max(x, axis=1, keepdims=True)
    s = rowtot
    for k in (1, 2, 4, 8, 16, 32, 64, 128):
        s = s + jnp.where(row[:, 0:1] >= k, pltpu.roll(s, k, 0), 0)
    dest = (s - rowtot) + (x - eli)
    dest_ref[:] = jnp.where(elig, dest, _TRASH + flat)


def _sc_compact_body(p0, p1, p2, p3, p4, p5, p6, dest_hbm,
                     o0, o1, o2, o3, o4, o5, o6,
                     v0, v1, v2, v3, v4, v5, v6,
                     i0, i1, i2, i3, i4, i5, i6, i7, i8, i9,
                     s0, s1, s2, s3, s4, s5, s6, sem):
    cid = lax.axis_index("c")
    sid = lax.axis_index("s")
    active = cid == 0
    base = sid * _PER
    planes = (p0, p1, p2, p3, p4, p5, p6)
    outs = (o0, o1, o2, o3, o4, o5, o6)
    vals = (v0, v1, v2, v3, v4, v5, v6)
    shared = (s0, s1, s2, s3, s4, s5, s6)
    idxs = (i0, i1, i2, i3, i4, i5, i6, i7, i8, i9)
    nch = _PER // _CHK
    slc = _TRASH // 16

    @pl.when(active)
    def _():
        loads = [pltpu.async_copy(dest_hbm.at[pl.ds(base + j * _CHK, _CHK)],
                                  idxs[j], sem) for j in range(nch)]
        loads += [pltpu.async_copy(planes[k].at[pl.ds(base, _PER)], vals[k],
                                   sem) for k in range(7)]
        for h in loads:
            h.wait()
        stores = [pltpu.async_copy(vals[k].at[pl.ds(j * _CHK, _CHK)],
                                   shared[k].at[idxs[j]], sem)
                  for k in range(7) for j in range(nch)]
        for h in stores:
            h.wait()

    plsc.subcore_barrier()

    @pl.when(active)
    def _():
        outc = [pltpu.async_copy(shared[k].at[pl.ds(sid * slc, slc)],
                                 outs[k].at[pl.ds(sid * slc, slc)], sem)
                for k in range(7)]
        for h in outc:
            h.wait()


def _sc_compact(planes, dest):
    mesh = plsc.VectorSubcoreMesh(core_axis_name="c", subcore_axis_name="s")
    f = pl.kernel(
        _sc_compact_body,
        mesh=mesh,
        out_type=[jax.ShapeDtypeStruct((_TRASH,), jnp.float32)] * 7,
        scratch_types=[pltpu.VMEM((_PER,), jnp.float32)] * 7
        + [pltpu.VMEM((_CHK,), jnp.int32)] * (_PER // _CHK)
        + [pltpu.VMEM_SHARED((_CAP,), jnp.float32)] * 7
        + [pltpu.SemaphoreType.DMA],
    )
    return f(*[planes[k] for k in range(7)], dest)


def _nms_body(scores_ref, coords_ref, out_ref, ws_ref, vols_ref):
    R = scores_ref.shape[0]
    NCH = R // _SUB
    big = jnp.int32(2 ** 30)
    ws_ref[:] = scores_ref[:]
    vols_ref[:] = ((coords_ref[3] - coords_ref[0])
                   * (coords_ref[4] - coords_ref[1])
                   * (coords_ref[5] - coords_ref[2]))

    ii = (lax.broadcasted_iota(jnp.int32, (_SUB, _LANES), 0) * _LANES
          + lax.broadcasted_iota(jnp.int32, (_SUB, _LANES), 1))
    lane = lax.broadcasted_iota(jnp.int32, (1, _LANES), 1)

    def comb_ordered(acc, v, i):
        if acc is None:
            return (v, i)
        av, ai = acc
        return (jnp.maximum(av, v), jnp.where(av >= v, ai, i))

    def comb_lex(a, b):
        av, ai = a
        bv, bi = b
        c = (av > bv) | ((av == bv) & (ai < bi))
        return (jnp.where(c, av, bv), jnp.where(c, ai, bi))

    accs = [None, None]
    for c in range(NCH):
        w = ws_ref[pl.ds(c * _SUB, _SUB), :]
        accs[c & 1] = comb_ordered(accs[c & 1], w, ii + c * _SUB * _LANES)
    v0, i0 = comb_lex(accs[0], accs[1])
    m0 = jnp.max(v0)
    idx0 = jnp.min(jnp.where(v0 == m0, i0, big))

    def body(i, carry):
        v8, i8 = carry
        m = jnp.max(v8)
        empty = m == _NEG
        sel = jnp.min(jnp.where(v8 == m, i8, big))
        sel = jnp.where(empty, idx0, sel)
        sc = jnp.where(empty, m0, m)
        rrow = lax.shift_right_logical(sel, 7)
        lsel = lax.bitwise_and(sel, 127)
        lmask = lane == lsel

        def pick(k):
            rowv = coords_ref[k, pl.ds(rrow, 1), :]
            return jnp.max(jnp.where(lmask, rowv, _NEG),
                           axis=1, keepdims=True)

        y1 = pick(0)
        x1 = pick(1)
        z1 = pick(2)
        y2 = pick(3)
        x2 = pick(4)
        z2 = pick(5)
        vol1 = (y2 - y1) * (x2 - x1) * (z2 - z1)

        accs = [None, None]
        for c in range(NCH):
            sl = pl.ds(c * _SUB, _SUB)
            w = ws_ref[sl, :]
            b0 = coords_ref[0, sl, :]
            b1 = coords_ref[1, sl, :]
            b2 = coords_ref[2, sl, :]
            b3 = coords_ref[3, sl, :]
            b4 = coords_ref[4, sl, :]
            b5 = coords_ref[5, sl, :]
            vv = vols_ref[sl, :]
            inter = (jnp.maximum(jnp.minimum(y2, b3) - jnp.maximum(y1, b0), 0.0)
                     * jnp.maximum(jnp.minimum(x2, b4) - jnp.maximum(x1, b1), 0.0)
                     * jnp.maximum(jnp.minimum(z2, b5) - jnp.maximum(z1, b2), 0.0))
            iou = inter / ((vol1 + vv - inter) + 1e-8)
            neww = jnp.where(iou > _THR, _NEG, w)
            ws_ref[sl, :] = neww
            accs[c & 1] = comb_ordered(accs[c & 1], neww,
                                       ii + c * _SUB * _LANES)

        v = jnp.full((1, _LANES), 0.0, jnp.float32)
        for j, val in enumerate((y1, x1, z1, y2, x2, z2, sc)):
            v = jnp.where(lane == j, val, v)
        out_ref[pl.ds(i, 1), :] = v
        return comb_lex(accs[0], accs[1])

    lax.fori_loop(0, _K, body, (v0, i0))


@jax.jit
def kernel(boxes, scores):
    n = scores.shape[0]
    pad = _N - n
    scores_p = jnp.concatenate(
        [scores, jnp.full((pad,), _NEG, jnp.float32)])
    boxes_p = jnp.concatenate([boxes, jnp.zeros((pad, 6), jnp.float32)],
                              axis=0)
    planes = list(boxes_p.T) + [scores_p]

    dest_plane = pl.pallas_call(
        _thresh_body,
        out_shape=jax.ShapeDtypeStruct((_N // _LANES, _LANES), jnp.int32),
    )(scores_p.reshape(_N // _LANES, _LANES))

    comp = _sc_compact(planes, dest_plane.reshape(_N))
    keep = lax.broadcasted_iota(jnp.int32, (_TRASH,), 0) < _PRE
    scores_c = jnp.where(keep, comp[6], _NEG).reshape(
        _TRASH // _LANES, _LANES)
    coords_c = jnp.stack(
        [jnp.where(keep, comp[k], 0.0).reshape(
            _TRASH // _LANES, _LANES) for k in range(6)])

    out = pl.pallas_call(
        _nms_body,
        out_shape=jax.ShapeDtypeStruct((304, _LANES), jnp.float32),
        scratch_shapes=[
            pltpu.VMEM((_TRASH // _LANES, _LANES), jnp.float32),
            pltpu.VMEM((_TRASH // _LANES, _LANES), jnp.float32),
        ],
    )(scores_c, coords_c)
    return out[:_K, :7]


# speedup vs baseline: 4.9756x; 1.0310x over previous
import functools

import jax
import jax.numpy as jnp
from jax import lax
from jax.experimental import pallas as pl
from jax.experimental.pallas import tpu as pltpu
from jax.experimental.pallas import tpu_sc as plsc

_N = 20480
_PRE = 6000
_CAP = 6144 + 20480
_TRASH = 6144
_THR = 0.3
_K = 300
_NEG = float("-inf")
_LANES = 128
_SUB = 8
_NW = 32
_CHK = 128
_PER = _N // 16
_VEC = 16


def _thresh_body(scores_ref, dest_ref):
    R = scores_ref.shape[0]
    scores = scores_ref[:]
    sbits = lax.bitcast_convert_type(scores, jnp.int32)
    row = lax.broadcasted_iota(jnp.int32, (R, _LANES), 0)
    col = lax.broadcasted_iota(jnp.int32, (R, _LANES), 1)
    flat = row * _LANES + col

    def bs_val(_, lh):
        lo, hi = lh
        mid = lo + (hi - lo) // 2
        below = jnp.sum((sbits > mid).astype(jnp.int32)) < _PRE
        return (jnp.where(below, lo, mid + 1), jnp.where(below, mid, hi))

    tbits, _ = lax.fori_loop(0, 31, bs_val,
                             (jnp.int32(0), jnp.int32(0x3F800000)))
    gt = sbits > tbits
    eq = sbits == tbits
    need = _PRE - jnp.sum(gt.astype(jnp.int32))

    def bs_idx(_, lh):
        lo, hi = lh
        mid = lo + (hi - lo) // 2
        ok = jnp.sum((eq & (flat < mid)).astype(jnp.int32)) >= need
        return (jnp.where(ok, lo, mid + 1), jnp.where(ok, mid, hi))

    cut, _ = lax.fori_loop(0, 16, bs_idx,
                           (jnp.int32(0), jnp.int32(R * _LANES)))
    elig = gt | (eq & (flat < cut))

    eli = jnp.where(elig, 1, 0).astype(jnp.int32)
    x = eli
    for k in (1, 2, 4, 8, 16, 32, 64):
        x = x + jnp.where(col >= k, pltpu.roll(x, k, 1), 0)
    rowtot = jnp.max(x, axis=1, keepdims=True)
    s = rowtot
    for k in (1, 2, 4, 8, 16, 32, 64, 128):
        s = s + jnp.where(row[:, 0:1] >= k, pltpu.roll(s, k, 0), 0)
    dest = (s - rowtot) + (x - eli)
    dest_ref[:] = jnp.where(elig, dest, _TRASH + flat)


def _sc_compact_body(p0, p1, p2, p3, p4, p5, p6, dest_hbm,
                     o0, o1, o2, o3, o4, o5, o6,
                     v0, v1, v2, v3, v4, v5, v6,
                     i0, i1, i2, i3, i4, i5, i6, i7, i8, i9,
                     s0, s1, s2, s3, s4, s5, s6, sem):
    cid = lax.axis_index("c")
    sid = lax.axis_index("s")
    active = cid == 0
    base = sid * _PER
    planes = (p0, p1, p2, p3, p4, p5, p6)
    outs = (o0, o1, o2, o3, o4, o5, o6)
    vals = (v0, v1, v2, v3, v4, v5, v6)
    shared = (s0, s1, s2, s3, s4, s5, s6)
    idxs = (i0, i1, i2, i3, i4, i5, i6, i7, i8, i9)
    nch = _PER // _CHK
    slc = _TRASH // 16

    @pl.when(active)
    def _():
        loads = [pltpu.async_copy(dest_hbm.at[pl.ds(base + j * _CHK, _CHK)],
                                  idxs[j], sem) for j in range(nch)]
        loads += [pltpu.async_copy(planes[k].at[pl.ds(base, _PER)], vals[k],
                                   sem) for k in range(7)]
        for h in loads:
            h.wait()
        stores = [pltpu.async_copy(vals[k].at[pl.ds(j * _CHK, _CHK)],
                                   shared[k].at[idxs[j]], sem)
                  for k in range(7) for j in range(nch)]
        for h in stores:
            h.wait()

    plsc.subcore_barrier()

    @pl.when(active)
    def _():
        outc = [pltpu.async_copy(shared[k].at[pl.ds(sid * slc, slc)],
                                 outs[k].at[pl.ds(sid * slc, slc)], sem)
                for k in range(7)]
        for h in outc:
            h.wait()


def _sc_compact(planes, dest):
    mesh = plsc.VectorSubcoreMesh(core_axis_name="c", subcore_axis_name="s")
    f = pl.kernel(
        _sc_compact_body,
        mesh=mesh,
        out_type=[jax.ShapeDtypeStruct((_TRASH,), jnp.float32)] * 7,
        scratch_types=[pltpu.VMEM((_PER,), jnp.float32)] * 7
        + [pltpu.VMEM((_CHK,), jnp.int32)] * (_PER // _CHK)
        + [pltpu.VMEM_SHARED((_CAP,), jnp.float32)] * 7
        + [pltpu.SemaphoreType.DMA],
    )
    return f(*[planes[k] for k in range(7)], dest)


def _nms_body(scores_ref, c0r, c1r, c2r, c3r, c4r, c5r,
              out_ref, ws_ref, vols_ref):
    R = scores_ref.shape[0]
    NCH = R // _SUB
    big = jnp.int32(2 ** 30)
    crefs = (c0r, c1r, c2r, c3r, c4r, c5r)
    rowf = lax.broadcasted_iota(jnp.int32, (R, _LANES), 0)
    colf = lax.broadcasted_iota(jnp.int32, (R, _LANES), 1)
    flatf = rowf * _LANES + colf
    ws_ref[:] = jnp.where(flatf < _PRE, scores_ref[:], _NEG)
    vols_ref[:] = ((c3r[:] - c0r[:]) * (c4r[:] - c1r[:]) * (c5r[:] - c2r[:]))

    ii = (lax.broadcasted_iota(jnp.int32, (_SUB, _LANES), 0) * _LANES
          + lax.broadcasted_iota(jnp.int32, (_SUB, _LANES), 1))
    lane = lax.broadcasted_iota(jnp.int32, (1, _LANES), 1)

    def comb_ordered(acc, v, i):
        if acc is None:
            return (v, i)
        av, ai = acc
        return (jnp.maximum(av, v), jnp.where(av >= v, ai, i))

    def comb_lex(a, b):
        av, ai = a
        bv, bi = b
        c = (av > bv) | ((av == bv) & (ai < bi))
        return (jnp.where(c, av, bv), jnp.where(c, ai, bi))

    accs = [None, None]
    for c in range(NCH):
        w = ws_ref[pl.ds(c * _SUB, _SUB), :]
        accs[c & 1] = comb_ordered(accs[c & 1], w, ii + c * _SUB * _LANES)
    v0, i0 = comb_lex(accs[0], accs[1])
    m0 = jnp.max(v0)
    idx0 = jnp.min(jnp.where(v0 == m0, i0, big))

    def body(i, carry):
        v8, i8 = carry
        m = jnp.max(v8)
        empty = m == _NEG
        sel = jnp.min(jnp.where(v8 == m, i8, big))
        sel = jnp.where(empty, idx0, sel)
        sc = jnp.where(empty, m0, m)
        rrow = lax.shift_right_logical(sel, 7)
        lsel = lax.bitwise_and(sel, 127)
        lmask = lane == lsel

        def pick(k):
            rowv = crefs[k][pl.ds(rrow, 1), :]
            return jnp.max(jnp.where(lmask, rowv, _NEG),
                           axis=1, keepdims=True)

        y1 = pick(0)
        x1 = pick(1)
        z1 = pick(2)
        y2 = pick(3)
        x2 = pick(4)
        z2 = pick(5)
        vol1 = (y2 - y1) * (x2 - x1) * (z2 - z1)

        accs = [None, None]
        for c in range(NCH):
            sl = pl.ds(c * _SUB, _SUB)
            w = ws_ref[sl, :]
            b0 = c0r[sl, :]
            b1 = c1r[sl, :]
            b2 = c2r[sl, :]
            b3 = c3r[sl, :]
            b4 = c4r[sl, :]
            b5 = c5r[sl, :]
            vv = vols_ref[sl, :]
            inter = (jnp.maximum(jnp.minimum(y2, b3) - jnp.maximum(y1, b0), 0.0)
                     * jnp.maximum(jnp.minimum(x2, b4) - jnp.maximum(x1, b1), 0.0)
                     * jnp.maximum(jnp.minimum(z2, b5) - jnp.maximum(z1, b2), 0.0))
            iou = inter / ((vol1 + vv - inter) + 1e-8)
            neww = jnp.where(iou > _THR, _NEG, w)
            ws_ref[sl, :] = neww
            accs[c & 1] = comb_ordered(accs[c & 1], neww,
                                       ii + c * _SUB * _LANES)

        v = jnp.full((1, _LANES), 0.0, jnp.float32)
        for j, val in enumerate((y1, x1, z1, y2, x2, z2, sc)):
            v = jnp.where(lane == j, val, v)
        out_ref[pl.ds(i, 1), :] = v
        return comb_lex(accs[0], accs[1])

    lax.fori_loop(0, _K, body, (v0, i0))


@jax.jit
def kernel(boxes, scores):
    n = scores.shape[0]
    pad = _N - n
    scores_p = jnp.concatenate(
        [scores, jnp.full((pad,), _NEG, jnp.float32)])
    boxes_p = jnp.concatenate([boxes, jnp.zeros((pad, 6), jnp.float32)],
                              axis=0)
    dest_plane = pl.pallas_call(
        _thresh_body,
        out_shape=jax.ShapeDtypeStruct((_N // _LANES, _LANES), jnp.int32),
    )(scores_p.reshape(_N // _LANES, _LANES))

    planes = list(boxes_p.T) + [scores_p]
    comp = _sc_compact(planes, dest_plane.reshape(_N))
    shaped = [comp[k].reshape(_TRASH // _LANES, _LANES) for k in range(7)]

    out = pl.pallas_call(
        _nms_body,
        out_shape=jax.ShapeDtypeStruct((304, _LANES), jnp.float32),
        scratch_shapes=[
            pltpu.VMEM((_TRASH // _LANES, _LANES), jnp.float32),
            pltpu.VMEM((_TRASH // _LANES, _LANES), jnp.float32),
        ],
    )(shaped[6], *shaped[:6])
    return out[:_K, :7]


# speedup vs baseline: 4.9892x; 1.0027x over previous
import jax
import jax.numpy as jnp
from jax import lax
from jax.experimental import pallas as pl
from jax.experimental.pallas import tpu as pltpu
from jax.experimental.pallas import tpu_sc as plsc

_N = 20480
_PRE = 6000
_CAP = 6144 + 20480
_TRASH = 6144
_THR = 0.3
_K = 300
_NEG = float("-inf")
_LANES = 128
_SUB = 8
_NW = 32
_CHK = 128
_PER = _N // 16
_VEC = 16


def _thresh_body(scores_ref, dest_ref):
    R = scores_ref.shape[0]
    scores = scores_ref[:]
    sbits = lax.bitcast_convert_type(scores, jnp.int32)
    row = lax.broadcasted_iota(jnp.int32, (R, _LANES), 0)
    col = lax.broadcasted_iota(jnp.int32, (R, _LANES), 1)
    flat = row * _LANES + col

    def bs_val(_, lh):
        lo, hi = lh
        mid = lo + (hi - lo) // 2
        below = jnp.sum((sbits > mid).astype(jnp.int32)) < _PRE
        return (jnp.where(below, lo, mid + 1), jnp.where(below, mid, hi))

    tbits, _ = lax.fori_loop(0, 31, bs_val,
                             (jnp.int32(0), jnp.int32(0x3F800000)))
    gt = sbits > tbits
    eq = sbits == tbits
    need = _PRE - jnp.sum(gt.astype(jnp.int32))

    def bs_idx(_, lh):
        lo, hi = lh
        mid = lo + (hi - lo) // 2
        ok = jnp.sum((eq & (flat < mid)).astype(jnp.int32)) >= need
        return (jnp.where(ok, lo, mid + 1), jnp.where(ok, mid, hi))

    cut, _ = lax.fori_loop(0, 16, bs_idx,
                           (jnp.int32(0), jnp.int32(R * _LANES)))
    elig = gt | (eq & (flat < cut))

    eli = jnp.where(elig, 1, 0).astype(jnp.int32)
    x = eli
    for k in (1, 2, 4, 8, 16, 32, 64):
        x = x + jnp.where(col >= k, pltpu.roll(x, k, 1), 0)
    rowtot = jnp.max(x, axis=1, keepdims=True)
    s = rowtot
    for k in (1, 2, 4, 8, 16, 32, 64, 128):
        s = s + jnp.where(row[:, 0:1] >= k, pltpu.roll(s, k, 0), 0)
    dest = (s - rowtot) + (x - eli)
    dest_ref[:] = jnp.where(elig, dest, _TRASH + flat)


def _sc_compact_body(p0, p1, p2, p3, p4, p5, p6, dest_hbm,
                     o0, o1, o2, o3, o4, o5, o6,
                     v0, v1, v2, v3, v4, v5, v6,
                     i0, i1, i2, i3, i4, i5, i6, i7, i8, i9,
                     s0, s1, s2, s3, s4, s5, s6, sem):
    cid = lax.axis_index("c")
    sid = lax.axis_index("s")
    active = cid == 0
    base = sid * _PER
    planes = (p0, p1, p2, p3, p4, p5, p6)
    outs = (o0, o1, o2, o3, o4, o5, o6)
    vals = (v0, v1, v2, v3, v4, v5, v6)
    shared = (s0, s1, s2, s3, s4, s5, s6)
    idxs = (i0, i1, i2, i3, i4, i5, i6, i7, i8, i9)
    nch = _PER // _CHK
    slc = _TRASH // 16

    @pl.when(active)
    def _():
        loads = [pltpu.async_copy(dest_hbm.at[pl.ds(base + j * _CHK, _CHK)],
                                  idxs[j], sem) for j in range(nch)]
        loads += [pltpu.async_copy(planes[k].at[pl.ds(base, _PER)], vals[k],
                                   sem) for k in range(7)]
        for h in loads:
            h.wait()
        stores = [pltpu.async_copy(vals[k].at[pl.ds(j * _CHK, _CHK)],
                                   shared[k].at[idxs[j]], sem)
                  for k in range(7) for j in range(nch)]
        for h in stores:
            h.wait()

    plsc.subcore_barrier()

    @pl.when(active)
    def _():
        outc = [pltpu.async_copy(shared[k].at[pl.ds(sid * slc, slc)],
                                 outs[k].at[pl.ds(sid * slc, slc)], sem)
                for k in range(7)]
        for h in outc:
            h.wait()


def _sc_compact(planes, dest):
    mesh = plsc.VectorSubcoreMesh(core_axis_name="c", subcore_axis_name="s")
    f = pl.kernel(
        _sc_compact_body,
        mesh=mesh,
        out_type=[jax.ShapeDtypeStruct((_TRASH,), jnp.float32)] * 7,
        scratch_types=[pltpu.VMEM((_PER,), jnp.float32)] * 7
        + [pltpu.VMEM((_CHK,), jnp.int32)] * (_PER // _CHK)
        + [pltpu.VMEM_SHARED((_CAP,), jnp.float32)] * 7
        + [pltpu.SemaphoreType.DMA],
    )
    return f(*[planes[k] for k in range(7)], dest)


def _nms_body(scores_ref, c0r, c1r, c2r, c3r, c4r, c5r,
              out_ref, ws_ref, vols_ref):
    R = scores_ref.shape[0]
    NCH = R // _SUB
    big = jnp.int32(2 ** 30)
    crefs = (c0r, c1r, c2r, c3r, c4r, c5r)
    rowf = lax.broadcasted_iota(jnp.int32, (R, _LANES), 0)
    colf = lax.broadcasted_iota(jnp.int32, (R, _LANES), 1)
    flatf = rowf * _LANES + colf
    ws_ref[:] = jnp.where(flatf < _PRE, scores_ref[:], _NEG)
    vols_ref[:] = ((c3r[:] - c0r[:]) * (c4r[:] - c1r[:]) * (c5r[:] - c2r[:]))

    ii = (lax.broadcasted_iota(jnp.int32, (_SUB, _LANES), 0) * _LANES
          + lax.broadcasted_iota(jnp.int32, (_SUB, _LANES), 1))
    lane = lax.broadcasted_iota(jnp.int32, (1, _LANES), 1)

    def comb_ordered(acc, v, i):
        if acc is None:
            return (v, i)
        av, ai = acc
        return (jnp.maximum(av, v), jnp.where(av >= v, ai, i))

    def comb_lex(a, b):
        av, ai = a
        bv, bi = b
        c = (av > bv) | ((av == bv) & (ai < bi))
        return (jnp.where(c, av, bv), jnp.where(c, ai, bi))

    accs = [None, None]
    for c in range(NCH):
        w = ws_ref[pl.ds(c * _SUB, _SUB), :]
        accs[c & 1] = comb_ordered(accs[c & 1], w, ii + c * _SUB * _LANES)
    v0, i0 = comb_lex(accs[0], accs[1])
    m0 = jnp.max(v0)
    idx0 = jnp.min(jnp.where(v0 == m0, i0, big))

    def body(i, carry):
        v8, i8 = carry
        m = jnp.max(v8)
        empty = m == _NEG
        sel = jnp.min(jnp.where(v8 == m, i8, big))
        sel = jnp.where(empty, idx0, sel)
        sc = jnp.where(empty, m0, m)
        rrow = lax.shift_right_logical(sel, 7)
        lsel = lax.bitwise_and(sel, 127)
        lmask = lane == lsel

        def pick(k):
            rowv = crefs[k][pl.ds(rrow, 1), :]
            return jnp.max(jnp.where(lmask, rowv, _NEG),
                           axis=1, keepdims=True)

        y1 = pick(0)
        x1 = pick(1)
        z1 = pick(2)
        y2 = pick(3)
        x2 = pick(4)
        z2 = pick(5)
        vol1 = (y2 - y1) * (x2 - x1) * (z2 - z1)

        accs = [None, None]
        for c in range(NCH):
            sl = pl.ds(c * _SUB, _SUB)
            w = ws_ref[sl, :]
            b0 = c0r[sl, :]
            b1 = c1r[sl, :]
            b2 = c2r[sl, :]
            b3 = c3r[sl, :]
            b4 = c4r[sl, :]
            b5 = c5r[sl, :]
            vv = vols_ref[sl, :]
            inter = (jnp.maximum(jnp.minimum(y2, b3) - jnp.maximum(y1, b0), 0.0)
                     * jnp.maximum(jnp.minimum(x2, b4) - jnp.maximum(x1, b1), 0.0)
                     * jnp.maximum(jnp.minimum(z2, b5) - jnp.maximum(z1, b2), 0.0))
            iou = inter / ((vol1 + vv - inter) + 1e-8)
            neww = jnp.where(iou > _THR, _NEG, w)
            ws_ref[sl, :] = neww
            accs[c & 1] = comb_ordered(accs[c & 1], neww,
                                       ii + c * _SUB * _LANES)

        v = jnp.full((1, _LANES), 0.0, jnp.float32)
        for j, val in enumerate((y1, x1, z1, y2, x2, z2, sc)):
            v = jnp.where(lane == j, val, v)
        out_ref[pl.ds(i, 1), :] = v
        return comb_lex(accs[0], accs[1])

    lax.fori_loop(0, _K, body, (v0, i0))


@jax.jit
def kernel(boxes, scores):
    n = scores.shape[0]
    pad = _N - n
    scores_p = jnp.concatenate(
        [scores, jnp.full((pad,), _NEG, jnp.float32)])
    boxes_p = jnp.concatenate([boxes, jnp.zeros((pad, 6), jnp.float32)],
                              axis=0)
    dest_plane = pl.pallas_call(
        _thresh_body,
        out_shape=jax.ShapeDtypeStruct((_N // _LANES, _LANES), jnp.int32),
    )(scores_p.reshape(_N // _LANES, _LANES))

    planes = list(boxes_p.T) + [scores_p]
    comp = _sc_compact(planes, dest_plane.reshape(_N))
    shaped = [comp[k].reshape(_TRASH // _LANES, _LANES) for k in range(7)]

    out = pl.pallas_call(
        _nms_body,
        out_shape=jax.ShapeDtypeStruct((304, _LANES), jnp.float32),
        scratch_shapes=[
            pltpu.VMEM((_TRASH // _LANES, _LANES), jnp.float32),
            pltpu.VMEM((_TRASH // _LANES, _LANES), jnp.float32),
        ],
    )(shaped[6], *shaped[:6])
    return out[:_K, :7]
